# baseline (speedup 1.0000x reference)
import functools

import jax
import jax.numpy as jnp
from jax import lax
from jax.experimental import pallas as pl
from jax.experimental.pallas import tpu as pltpu
from jax.experimental.pallas import tpu_sc as plsc

_G = 64
_NW = 32
_N = 100000
_NPAD = 131072
_ROWS_W = 32
_SEGROWS = 80


def _sc_counts_kernel(batch_hbm, out_hbm, sbuf, obuf):
    cid = lax.axis_index("c")
    sid = lax.axis_index("s")
    wid = sid * 2 + cid

    pltpu.sync_copy(batch_hbm.at[pl.ds(wid * _ROWS_W, _ROWS_W)], sbuf)

    one = jnp.ones((16,), jnp.int32)
    zero = jnp.zeros((16,), jnp.int32)

    for g in range(4):
        def body(c, accs):
            j = c // 8
            off = (c % 8) * 16
            idx = sbuf[j, pl.ds(off, 16)]
            return tuple(
                accs[l] + jnp.where(idx == (g * 16 + l), one, zero)
                for l in range(16))

        accs = lax.fori_loop(
            0, _ROWS_W * 8, body,
            tuple(jnp.zeros((16,), jnp.int32) for _ in range(16)))
        for l in range(16):
            obuf[g * 16 + l, :] = accs[l].astype(jnp.float32)

    pltpu.sync_copy(obuf, out_hbm.at[wid])


def _segment_counts(batch_pad_2d):
    kern = functools.partial(
        pl.kernel,
        out_type=jax.ShapeDtypeStruct((_NW, _G, 16), jnp.float32),
        mesh=plsc.VectorSubcoreMesh(core_axis_name="c", subcore_axis_name="s"),
        scratch_types=[
            pltpu.VMEM((_ROWS_W, 128), jnp.int32),
            pltpu.VMEM((_G, 16), jnp.float32),
        ],
    )(_sc_counts_kernel)
    return kern(batch_pad_2d)


def _stats_kernel(b_ref, x_ref, sum_ref, sq_ref):
    i = pl.program_id(0)

    @pl.when(i == 0)
    def _():
        sum_ref[...] = jnp.zeros_like(sum_ref)
        sq_ref[...] = jnp.zeros_like(sq_ref)

    bb = b_ref[0, 0, :]
    xb = x_ref[...]
    onehot = (bb[:, None] == jax.lax.broadcasted_iota(
        jnp.int32, (bb.shape[0], _G), 1)).astype(jnp.float32)
    dn = (((0,), (0,)), ((), ()))
    sum_ref[...] += jax.lax.dot_general(
        onehot, xb, dn, preferred_element_type=jnp.float32)
    sq_ref[...] += jax.lax.dot_general(
        onehot, xb * xb, dn, preferred_element_type=jnp.float32)


def _norm_kernel(b_ref, x_ref, sum_ref, sq_ref, ct_ref, g_ref, be_ref,
                 al_ref, out_ref):
    bb = b_ref[0, 0, :]
    xb = x_ref[...]
    ct = jnp.maximum(jnp.sum(ct_ref[...], axis=(0, 2)), 1.0)
    inv_ct = (1.0 / ct)[:, None]
    mean = sum_ref[...] * inv_ct
    meansq = sq_ref[...] * inv_ct
    al = al_ref[0, 0]
    v = meansq - (al * (2.0 - al)) * mean * mean
    v = jnp.maximum(v, 0.0)
    scale = g_ref[...] / (jnp.sqrt(v) + 1e-5)
    bias = be_ref[...] - scale * (al * mean)
    onehot = (bb[:, None] == jax.lax.broadcasted_iota(
        jnp.int32, (bb.shape[0], _G), 1)).astype(jnp.float32)
    dn = (((1,), (0,)), ((), ()))
    sc_r = jax.lax.dot_general(
        onehot, scale, dn, preferred_element_type=jnp.float32)
    bi_r = jax.lax.dot_general(
        onehot, bias, dn, preferred_element_type=jnp.float32)
    out_ref[...] = sc_r * xb + bi_r


def kernel(x, batch, gamma, beta, alpha):
    n, hd = x.shape
    bn = 5000
    grid = n // bn
    batch_i32 = batch.astype(jnp.int32)
    batch3 = batch_i32.reshape(grid, 1, bn)
    gamma2 = gamma.reshape(1, hd)
    beta2 = beta.reshape(1, hd)
    alpha2 = alpha.reshape(1, 1)

    batch_pad = jnp.concatenate(
        [batch_i32, jnp.full((_NPAD - n,), _G, jnp.int32)]
    ).reshape(_NPAD // 128, 128)
    cts = _segment_counts(batch_pad)

    b_spec = pl.BlockSpec((1, 1, bn), lambda i: (i, 0, 0))
    x_spec = pl.BlockSpec((bn, hd), lambda i: (i, 0))
    g_spec = pl.BlockSpec((_G, hd), lambda i: (0, 0))
    ct_spec = pl.BlockSpec((_NW, _G, 16), lambda i: (0, 0, 0))

    sums, sqs = pl.pallas_call(
        _stats_kernel,
        grid=(grid,),
        in_specs=[b_spec, x_spec],
        out_specs=[g_spec, g_spec],
        out_shape=[
            jax.ShapeDtypeStruct((_G, hd), jnp.float32),
            jax.ShapeDtypeStruct((_G, hd), jnp.float32),
        ],
    )(batch3, x)

    out = pl.pallas_call(
        _norm_kernel,
        grid=(grid,),
        in_specs=[b_spec, x_spec, g_spec, g_spec, ct_spec,
                  pl.BlockSpec((1, hd), lambda i: (0, 0)),
                  pl.BlockSpec((1, hd), lambda i: (0, 0)),
                  pl.BlockSpec((1, 1), lambda i: (0, 0))],
        out_specs=x_spec,
        out_shape=jax.ShapeDtypeStruct((n, hd), jnp.float32),
    )(batch3, x, sums, sqs, cts, gamma2, beta2, alpha2)
    return out


# speedup vs baseline: 1.0387x; 1.0387x over previous
import jax
import jax.numpy as jnp
from jax.experimental import pallas as pl
from jax.experimental.pallas import tpu as pltpu

_G = 64


def _fused_kernel(b_ref, x_ref, g_ref, be_ref, al_ref, out_ref,
                  sum_ref, sq_ref, ct_ref, *, nblk):
    i = pl.program_id(0)

    @pl.when(i == 0)
    def _():
        sum_ref[...] = jnp.zeros_like(sum_ref)
        sq_ref[...] = jnp.zeros_like(sq_ref)
        ct_ref[...] = jnp.zeros_like(ct_ref)

    bb = b_ref[0, 0, :]
    xb = x_ref[...]
    onehot = (bb[:, None] == jax.lax.broadcasted_iota(
        jnp.int32, (bb.shape[0], _G), 1)).astype(jnp.float32)

    @pl.when(i < nblk)
    def _():
        dn = (((0,), (0,)), ((), ()))
        sum_ref[...] += jax.lax.dot_general(
            onehot, xb, dn, preferred_element_type=jnp.float32)
        sq_ref[...] += jax.lax.dot_general(
            onehot, xb * xb, dn, preferred_element_type=jnp.float32)
        ct_ref[0, :] += jnp.sum(onehot, axis=0)

    @pl.when(i >= nblk)
    def _():
        ct = jnp.maximum(ct_ref[0, :], 1.0)
        inv_ct = (1.0 / ct)[:, None]
        mean = sum_ref[...] * inv_ct
        meansq = sq_ref[...] * inv_ct
        al = al_ref[0, 0]
        v = meansq - (al * (2.0 - al)) * mean * mean
        v = jnp.maximum(v, 0.0)
        scale = g_ref[...] / (jnp.sqrt(v) + 1e-5)
        bias = be_ref[...] - scale * (al * mean)
        dn = (((1,), (0,)), ((), ()))
        sc_r = jax.lax.dot_general(
            onehot, scale, dn, preferred_element_type=jnp.float32)
        bi_r = jax.lax.dot_general(
            onehot, bias, dn, preferred_element_type=jnp.float32)
        out_ref[...] = sc_r * xb + bi_r


def kernel(x, batch, gamma, beta, alpha):
    n, hd = x.shape
    bn = 4000
    nblk = n // bn
    batch3 = batch.astype(jnp.int32).reshape(nblk, 1, bn)
    gamma2 = gamma.reshape(1, hd)
    beta2 = beta.reshape(1, hd)
    alpha2 = alpha.reshape(1, 1)

    blk = lambda i: i % nblk
    import functools
    body = functools.partial(_fused_kernel, nblk=nblk)

    out = pl.pallas_call(
        body,
        grid=(2 * nblk,),
        in_specs=[
            pl.BlockSpec((1, 1, bn), lambda i: (i % nblk, 0, 0)),
            pl.BlockSpec((bn, hd), lambda i: (i % nblk, 0)),
            pl.BlockSpec((1, hd), lambda i: (0, 0)),
            pl.BlockSpec((1, hd), lambda i: (0, 0)),
            pl.BlockSpec((1, 1), lambda i: (0, 0)),
        ],
        out_specs=pl.BlockSpec(
            (bn, hd), lambda i: (jnp.maximum(i - nblk, 0), 0)),
        out_shape=jax.ShapeDtypeStruct((n, hd), jnp.float32),
        scratch_shapes=[
            pltpu.VMEM((_G, hd), jnp.float32),
            pltpu.VMEM((_G, hd), jnp.float32),
            pltpu.VMEM((1, _G), jnp.float32),
        ],
    )(batch3, x, gamma2, beta2, alpha2)
    return out


# speedup vs baseline: 1.1030x; 1.0619x over previous
import jax
import jax.numpy as jnp
from jax.experimental import pallas as pl

_G = 64


def _stats_kernel(b_ref, x_ref, sum_ref, sq_ref, ct_ref):
    i = pl.program_id(0)

    @pl.when(i == 0)
    def _():
        sum_ref[...] = jnp.zeros_like(sum_ref)
        sq_ref[...] = jnp.zeros_like(sq_ref)
        ct_ref[...] = jnp.zeros_like(ct_ref)

    bb = b_ref[0, 0, :]
    xb = x_ref[...]
    onehot = (bb[:, None] == jax.lax.broadcasted_iota(
        jnp.int32, (bb.shape[0], _G), 1)).astype(jnp.float32)
    dn = (((0,), (0,)), ((), ()))
    sum_ref[...] += jax.lax.dot_general(
        onehot, xb, dn, preferred_element_type=jnp.float32)
    sq_ref[...] += jax.lax.dot_general(
        onehot, xb * xb, dn, preferred_element_type=jnp.float32)
    ct_ref[0, :] += jnp.sum(onehot, axis=0)


def _norm_kernel(b_ref, x_ref, sum_ref, sq_ref, ct_ref, g_ref, be_ref,
                 al_ref, out_ref):
    bb = b_ref[0, 0, :]
    xb = x_ref[...]
    ct = jnp.maximum(ct_ref[0, :], 1.0)
    inv_ct = (1.0 / ct)[:, None]
    mean = sum_ref[...] * inv_ct
    meansq = sq_ref[...] * inv_ct
    al = al_ref[0, 0]
    v = meansq - (al * (2.0 - al)) * mean * mean
    v = jnp.maximum(v, 0.0)
    scale = g_ref[...] / (jnp.sqrt(v) + 1e-5)
    bias = be_ref[...] - scale * (al * mean)
    onehot = (bb[:, None] == jax.lax.broadcasted_iota(
        jnp.int32, (bb.shape[0], _G), 1)).astype(jnp.float32)
    dn = (((1,), (0,)), ((), ()))
    sc_r = jax.lax.dot_general(
        onehot, scale, dn, preferred_element_type=jnp.float32)
    bi_r = jax.lax.dot_general(
        onehot, bias, dn, preferred_element_type=jnp.float32)
    out_ref[...] = sc_r * xb + bi_r


def kernel(x, batch, gamma, beta, alpha):
    n, hd = x.shape
    bn1 = 10000
    bn2 = 5000
    grid1 = n // bn1
    grid2 = n // bn2
    batch_i32 = batch.astype(jnp.int32)
    batch1 = batch_i32.reshape(grid1, 1, bn1)
    batch2 = batch_i32.reshape(grid2, 1, bn2)
    gamma2 = gamma.reshape(1, hd)
    beta2 = beta.reshape(1, hd)
    alpha2 = alpha.reshape(1, 1)

    g_spec = pl.BlockSpec((_G, hd), lambda i: (0, 0))
    ct_spec = pl.BlockSpec((1, _G), lambda i: (0, 0))

    sums, sqs, cts = pl.pallas_call(
        _stats_kernel,
        grid=(grid1,),
        in_specs=[pl.BlockSpec((1, 1, bn1), lambda i: (i, 0, 0)),
                  pl.BlockSpec((bn1, hd), lambda i: (i, 0))],
        out_specs=[g_spec, g_spec, ct_spec],
        out_shape=[
            jax.ShapeDtypeStruct((_G, hd), jnp.float32),
            jax.ShapeDtypeStruct((_G, hd), jnp.float32),
            jax.ShapeDtypeStruct((1, _G), jnp.float32),
        ],
    )(batch1, x)

    x_spec2 = pl.BlockSpec((bn2, hd), lambda i: (i, 0))
    out = pl.pallas_call(
        _norm_kernel,
        grid=(grid2,),
        in_specs=[pl.BlockSpec((1, 1, bn2), lambda i: (i, 0, 0)),
                  x_spec2, g_spec, g_spec, ct_spec,
                  pl.BlockSpec((1, hd), lambda i: (0, 0)),
                  pl.BlockSpec((1, hd), lambda i: (0, 0)),
                  pl.BlockSpec((1, 1), lambda i: (0, 0))],
        out_specs=x_spec2,
        out_shape=jax.ShapeDtypeStruct((n, hd), jnp.float32),
    )(batch2, x, sums, sqs, cts, gamma2, beta2, alpha2)
    return out
